# baseline (speedup 1.0000x reference)
import functools

import jax
import jax.numpy as jnp
from jax import lax
from jax.experimental import pallas as pl
from jax.experimental.pallas import tpu as pltpu
from jax.experimental.pallas import tpu_sc as plsc

N_NODES = 10000
N_EDGES = 320000
D = 128

NC = 2
NS = 16
NW = NC * NS
E_PER_W = N_EDGES // NW
CHUNK = 80
NCHUNK = E_PER_W // CHUNK
NBUF = 5
ROWS_PER_TILE = N_NODES // NS
ZROWS = 125

_mesh = plsc.VectorSubcoreMesh(core_axis_name="c", subcore_axis_name="s")
_sc_params = pltpu.CompilerParams(
    needs_layout_passes=False, use_tc_tiling_on_sc=False)


def _deg_body(dst_hbm, ew_hbm, out_hbm, deg_l, dst_v, ew_v):
    cid = lax.axis_index("c")
    sid = lax.axis_index("s")
    wid = sid * NC + cid

    def zero(j, _):
        deg_l[pl.ds(j * 16, 16)] = jnp.zeros((16,), jnp.float32)
        return _

    lax.fori_loop(0, N_NODES // 16, zero, None)

    base = wid * E_PER_W
    pltpu.sync_copy(dst_hbm.at[pl.ds(base, E_PER_W)], dst_v)
    pltpu.sync_copy(ew_hbm.at[pl.ds(base, E_PER_W)], ew_v)

    def acc(j, _):
        d16 = dst_v[pl.ds(j * 16, 16)]
        w16 = ew_v[pl.ds(j * 16, 16)]
        plsc.addupdate_scatter(deg_l, [d16], w16)
        return _

    lax.fori_loop(0, E_PER_W // 16, acc, None)
    pltpu.sync_copy(deg_l, out_hbm.at[wid])


_deg_call = pl.kernel(
    _deg_body,
    out_type=jax.ShapeDtypeStruct((NW, N_NODES), jnp.float32),
    mesh=_mesh,
    scratch_types=[
        pltpu.VMEM((N_NODES,), jnp.float32),
        pltpu.VMEM((E_PER_W,), jnp.int32),
        pltpu.VMEM((E_PER_W,), jnp.float32),
    ],
    compiler_params=_sc_params,
)


DH = D // 2


def _agg_body(hs_hbm, srclo_hbm, srchi_hbm, dst_hbm, ew_hbm, out_hbm,
              srclo_v, srchi_v, dst_v, ew_v, rows_v, zbuf, acc_sh, sem, ssem):
    cid = lax.axis_index("c")
    sid = lax.axis_index("s")
    wid = sid * NC + cid

    pltpu.sync_copy(srclo_hbm.at[wid], srclo_v)
    pltpu.sync_copy(srchi_hbm.at[wid], srchi_v)
    pltpu.sync_copy(dst_hbm.at[wid], dst_v)
    pltpu.sync_copy(ew_hbm.at[wid], ew_v)

    def zrow(r, _):
        for d4 in range(DH // 16):
            zbuf[r, pl.ds(d4 * 16, 16)] = jnp.zeros((16,), jnp.float32)
        return _

    lax.fori_loop(0, ZROWS, zrow, None)

    for half in range(2):
        sv = (srclo_v, srchi_v)[half]
        for b in range(NBUF):
            pltpu.async_copy(hs_hbm.at[sv.at[pl.ds(b * CHUNK, CHUNK)]],
                             rows_v.at[b], sem.at[b])
        for k in range(ROWS_PER_TILE // ZROWS):
            pltpu.sync_copy(
                zbuf, acc_sh.at[pl.ds(sid * ROWS_PER_TILE + k * ZROWS, ZROWS)])
        plsc.subcore_barrier()

        def outer(go, _):
            g0 = go * NBUF
            for b in range(NBUF):
                ch = g0 + b
                pltpu.make_async_copy(
                    hs_hbm.at[sv.at[pl.ds(ch * CHUNK, CHUNK)]],
                    rows_v.at[b], sem.at[b]).wait()

                ebase = jnp.full((16,), ch * CHUNK, jnp.int32)

                @plsc.parallel_loop(0, CHUNK, 1, unroll=5)
                def _scale(i):
                    ewb = plsc.load_gather(ew_v, [ebase + i])
                    for d4 in range(DH // 16):
                        rows_v[b, i, pl.ds(d4 * 16, 16)] = (
                            rows_v[b, i, pl.ds(d4 * 16, 16)] * ewb)
                pltpu.async_copy(rows_v.at[b], acc_sh.at[dst_v.at[ch]],
                                 ssem.at[b], add=True)
                nxt = ch + 2
                b2 = (b + 2) % NBUF

                @pl.when(jnp.logical_and(nxt >= NBUF, nxt < NCHUNK))
                def _prefetch():
                    pltpu.make_async_copy(
                        hs_hbm.at[sv.at[pl.ds(nxt * CHUNK, CHUNK)]],
                        rows_v.at[b2], ssem.at[b2]).wait()
                    pltpu.async_copy(
                        hs_hbm.at[sv.at[pl.ds(nxt * CHUNK, CHUNK)]],
                        rows_v.at[b2], sem.at[b2])
            return _

        lax.fori_loop(0, NCHUNK // NBUF, outer, None)
        for b in range(NBUF):
            pltpu.make_async_copy(
                hs_hbm.at[sv.at[pl.ds(0, CHUNK)]], rows_v.at[b],
                ssem.at[b]).wait()
        plsc.subcore_barrier()

        pltpu.sync_copy(
            acc_sh.at[pl.ds(sid * ROWS_PER_TILE, ROWS_PER_TILE)],
            out_hbm.at[cid, pl.ds(sid * ROWS_PER_TILE, ROWS_PER_TILE),
                       pl.ds(half * DH, DH)])
        plsc.subcore_barrier()


_agg_call = pl.kernel(
    _agg_body,
    out_type=jax.ShapeDtypeStruct((NC, N_NODES, D), jnp.float32),
    mesh=_mesh,
    scratch_types=[
        pltpu.VMEM((E_PER_W,), jnp.int32),
        pltpu.VMEM((E_PER_W,), jnp.int32),
        pltpu.VMEM((NCHUNK, CHUNK), jnp.int32),
        pltpu.VMEM((E_PER_W,), jnp.float32),
        pltpu.VMEM((NBUF, CHUNK, DH), jnp.float32),
        pltpu.VMEM((ZROWS, DH), jnp.float32),
        pltpu.VMEM_SHARED((N_NODES, DH), jnp.float32),
        pltpu.SemaphoreType.DMA((NBUF,)),
        pltpu.SemaphoreType.DMA((NBUF,)),
    ],
    compiler_params=_sc_params,
)


_RB = 2000
_GRID = N_NODES // _RB


def _stage1_body(degT_ref, x_ref, w1_ref, h1_ref, hs_ref):
    deg = 1.0 + jnp.sum(degT_ref[...], axis=1, keepdims=True)
    dinv = lax.rsqrt(deg)
    h = jnp.dot(x_ref[...], w1_ref[...], preferred_element_type=jnp.float32)
    h1_ref[...] = h
    hs_ref[...] = h * dinv


def _stage1(degT, x, W1):
    return pl.pallas_call(
        _stage1_body,
        grid=(_GRID,),
        in_specs=[
            pl.BlockSpec((_RB, NW), lambda i: (i, 0)),
            pl.BlockSpec((_RB, D), lambda i: (i, 0)),
            pl.BlockSpec((D, D), lambda i: (0, 0)),
        ],
        out_specs=[
            pl.BlockSpec((_RB, D), lambda i: (i, 0)),
            pl.BlockSpec((_RB, D), lambda i: (i, 0)),
        ],
        out_shape=[
            jax.ShapeDtypeStruct((N_NODES, D), jnp.float32),
            jax.ShapeDtypeStruct((N_NODES, D), jnp.float32),
        ],
    )(degT, x, W1)


def _stage2_body(acc_ref, h1_ref, degT_ref, b1_ref, w2_ref,
                 h2_ref, hs_ref):
    dv = lax.rsqrt(1.0 + jnp.sum(degT_ref[...], axis=1, keepdims=True))
    a = acc_ref[0] + acc_ref[1]
    out1 = dv * a + dv * dv * h1_ref[...] + b1_ref[...]
    r = jnp.maximum(out1, 0.0)
    h2 = jnp.dot(r, w2_ref[...], preferred_element_type=jnp.float32)
    h2_ref[...] = h2
    hs_ref[...] = h2 * dv


def _stage2(acc1, h1, degT, b1, W2):
    return pl.pallas_call(
        _stage2_body,
        grid=(_GRID,),
        in_specs=[
            pl.BlockSpec((NC, _RB, D), lambda i: (0, i, 0)),
            pl.BlockSpec((_RB, D), lambda i: (i, 0)),
            pl.BlockSpec((_RB, NW), lambda i: (i, 0)),
            pl.BlockSpec((1, D), lambda i: (0, 0)),
            pl.BlockSpec((D, D), lambda i: (0, 0)),
        ],
        out_specs=[
            pl.BlockSpec((_RB, D), lambda i: (i, 0)),
            pl.BlockSpec((_RB, D), lambda i: (i, 0)),
        ],
        out_shape=[
            jax.ShapeDtypeStruct((N_NODES, D), jnp.float32),
            jax.ShapeDtypeStruct((N_NODES, D), jnp.float32),
        ],
    )(acc1, h1, degT, b1, W2)


def _stage3_body(acc_ref, h2_ref, degT_ref, b2_ref, out_ref):
    dv = lax.rsqrt(1.0 + jnp.sum(degT_ref[...], axis=1, keepdims=True))
    a = acc_ref[0] + acc_ref[1]
    out_ref[...] = dv * a + dv * dv * h2_ref[...] + b2_ref[...]


def _stage3(acc2, h2, degT, b2):
    return pl.pallas_call(
        _stage3_body,
        grid=(_GRID,),
        in_specs=[
            pl.BlockSpec((NC, _RB, D), lambda i: (0, i, 0)),
            pl.BlockSpec((_RB, D), lambda i: (i, 0)),
            pl.BlockSpec((_RB, NW), lambda i: (i, 0)),
            pl.BlockSpec((1, D), lambda i: (0, 0)),
        ],
        out_specs=pl.BlockSpec((_RB, D), lambda i: (i, 0)),
        out_shape=jax.ShapeDtypeStruct((N_NODES, D), jnp.float32),
    )(acc2, h2, degT, b2)


@jax.jit
def kernel(x, edge_index, edge_weight, W1, b1, W2, b2):
    src = edge_index[0].astype(jnp.int32)
    dst = edge_index[1].astype(jnp.int32)
    ew = edge_weight.astype(jnp.float32)

    srclo2 = (src * 2).reshape(NW, E_PER_W)
    srchi2 = (src * 2 + 1).reshape(NW, E_PER_W)
    dst3 = dst.reshape(NW, NCHUNK, CHUNK)
    ew2 = ew.reshape(NW, E_PER_W)

    deg_p = _deg_call(dst, ew)
    degT = deg_p.T
    h1, hs1 = _stage1(degT, x, W1)
    acc1 = _agg_call(hs1.reshape(2 * N_NODES, DH), srclo2, srchi2, dst3, ew2)
    h2, hs2 = _stage2(acc1, h1, degT, b1.reshape(1, D), W2)
    acc2 = _agg_call(hs2.reshape(2 * N_NODES, DH), srclo2, srchi2, dst3, ew2)
    return _stage3(acc2, h2, degT, b2.reshape(1, D))


# speedup vs baseline: 1.0802x; 1.0802x over previous
import functools

import jax
import jax.numpy as jnp
from jax import lax
from jax.experimental import pallas as pl
from jax.experimental.pallas import tpu as pltpu
from jax.experimental.pallas import tpu_sc as plsc

N_NODES = 10000
N_EDGES = 320000
D = 128

NC = 2
NS = 16
NW = NC * NS
E_PER_W = N_EDGES // NW
CHUNK = 125
NCHUNK = E_PER_W // CHUNK
NBUF = 5
ROWS_PER_TILE = N_NODES // NS
ZROWS = 125

_mesh = plsc.VectorSubcoreMesh(core_axis_name="c", subcore_axis_name="s")
_sc_params = pltpu.CompilerParams(
    needs_layout_passes=False, use_tc_tiling_on_sc=False)


def _deg_body(dst_hbm, ew_hbm, out_hbm, deg_l, dst_v, ew_v):
    cid = lax.axis_index("c")
    sid = lax.axis_index("s")
    wid = sid * NC + cid

    def zero(j, _):
        deg_l[pl.ds(j * 16, 16)] = jnp.zeros((16,), jnp.float32)
        return _

    lax.fori_loop(0, N_NODES // 16, zero, None)

    base = wid * E_PER_W
    pltpu.sync_copy(dst_hbm.at[pl.ds(base, E_PER_W)], dst_v)
    pltpu.sync_copy(ew_hbm.at[pl.ds(base, E_PER_W)], ew_v)

    def acc(j, _):
        d16 = dst_v[pl.ds(j * 16, 16)]
        w16 = ew_v[pl.ds(j * 16, 16)]
        plsc.addupdate_scatter(deg_l, [d16], w16)
        return _

    lax.fori_loop(0, E_PER_W // 16, acc, None)
    pltpu.sync_copy(deg_l, out_hbm.at[wid])


_deg_call = pl.kernel(
    _deg_body,
    out_type=jax.ShapeDtypeStruct((NW, N_NODES), jnp.float32),
    mesh=_mesh,
    scratch_types=[
        pltpu.VMEM((N_NODES,), jnp.float32),
        pltpu.VMEM((E_PER_W,), jnp.int32),
        pltpu.VMEM((E_PER_W,), jnp.float32),
    ],
    compiler_params=_sc_params,
)


DH = D // 2


def _agg_body(hs_lo_hbm, hs_hi_hbm, src_hbm, dst_hbm, ew_hbm, out_hbm,
              src_v, dst_v, ew_v, rows_v, zbuf, acc_sh, sem, ssem):
    cid = lax.axis_index("c")
    sid = lax.axis_index("s")
    wid = sid * NC + cid

    pltpu.sync_copy(src_hbm.at[wid], src_v)
    pltpu.sync_copy(dst_hbm.at[wid], dst_v)
    pltpu.sync_copy(ew_hbm.at[wid], ew_v)

    def zrow(r, _):
        for d4 in range(DH // 16):
            zbuf[r, pl.ds(d4 * 16, 16)] = jnp.zeros((16,), jnp.float32)
        return _

    lax.fori_loop(0, ZROWS, zrow, None)

    for half, hs_hbm in enumerate((hs_lo_hbm, hs_hi_hbm)):
        for b in range(NBUF):
            pltpu.async_copy(hs_hbm.at[src_v.at[b]], rows_v.at[b], sem.at[b])
        for k in range(ROWS_PER_TILE // ZROWS):
            pltpu.sync_copy(
                zbuf, acc_sh.at[pl.ds(sid * ROWS_PER_TILE + k * ZROWS, ZROWS)])
        plsc.subcore_barrier()

        def outer(go, _):
            g0 = go * NBUF
            for b in range(NBUF):
                ch = g0 + b
                pltpu.make_async_copy(
                    hs_hbm.at[src_v.at[ch]], rows_v.at[b], sem.at[b]).wait()

                ebase = jnp.full((16,), ch * CHUNK, jnp.int32)

                @plsc.parallel_loop(0, CHUNK, 1, unroll=5)
                def _scale(i):
                    ewb = plsc.load_gather(ew_v, [ebase + i])
                    for d4 in range(DH // 16):
                        rows_v[b, i, pl.ds(d4 * 16, 16)] = (
                            rows_v[b, i, pl.ds(d4 * 16, 16)] * ewb)
                pltpu.async_copy(rows_v.at[b], acc_sh.at[dst_v.at[ch]],
                                 ssem.at[b], add=True)
                nxt = ch + 2
                b2 = (b + 2) % NBUF

                @pl.when(jnp.logical_and(nxt >= NBUF, nxt < NCHUNK))
                def _prefetch():
                    pltpu.make_async_copy(
                        hs_hbm.at[src_v.at[nxt]], rows_v.at[b2],
                        ssem.at[b2]).wait()
                    pltpu.async_copy(
                        hs_hbm.at[src_v.at[nxt]], rows_v.at[b2], sem.at[b2])
            return _

        lax.fori_loop(0, NCHUNK // NBUF, outer, None)
        for b in range(NBUF):
            pltpu.make_async_copy(
                hs_hbm.at[src_v.at[0]], rows_v.at[b], ssem.at[b]).wait()
        plsc.subcore_barrier()

        pltpu.sync_copy(
            acc_sh.at[pl.ds(sid * ROWS_PER_TILE, ROWS_PER_TILE)],
            out_hbm.at[cid, pl.ds(sid * ROWS_PER_TILE, ROWS_PER_TILE),
                       pl.ds(half * DH, DH)])
        plsc.subcore_barrier()


_agg_call = pl.kernel(
    _agg_body,
    out_type=jax.ShapeDtypeStruct((NC, N_NODES, D), jnp.float32),
    mesh=_mesh,
    scratch_types=[
        pltpu.VMEM((NCHUNK, CHUNK), jnp.int32),
        pltpu.VMEM((NCHUNK, CHUNK), jnp.int32),
        pltpu.VMEM((E_PER_W,), jnp.float32),
        pltpu.VMEM((NBUF, CHUNK, DH), jnp.float32),
        pltpu.VMEM((ZROWS, DH), jnp.float32),
        pltpu.VMEM_SHARED((N_NODES, DH), jnp.float32),
        pltpu.SemaphoreType.DMA((NBUF,)),
        pltpu.SemaphoreType.DMA((NBUF,)),
    ],
    compiler_params=_sc_params,
)


_RB = 2000
_GRID = N_NODES // _RB


def _stage1_body(degT_ref, x_ref, w1_ref, h1_ref, hs_lo_ref, hs_hi_ref):
    deg = 1.0 + jnp.sum(degT_ref[...], axis=1, keepdims=True)
    dinv = lax.rsqrt(deg)
    h = jnp.dot(x_ref[...], w1_ref[...], preferred_element_type=jnp.float32)
    h1_ref[...] = h
    hs = h * dinv
    hs_lo_ref[...] = hs[:, :DH]
    hs_hi_ref[...] = hs[:, DH:]


def _stage1(degT, x, W1):
    return pl.pallas_call(
        _stage1_body,
        grid=(_GRID,),
        in_specs=[
            pl.BlockSpec((_RB, NW), lambda i: (i, 0)),
            pl.BlockSpec((_RB, D), lambda i: (i, 0)),
            pl.BlockSpec((D, D), lambda i: (0, 0)),
        ],
        out_specs=[
            pl.BlockSpec((_RB, D), lambda i: (i, 0)),
            pl.BlockSpec((_RB, DH), lambda i: (i, 0)),
            pl.BlockSpec((_RB, DH), lambda i: (i, 0)),
        ],
        out_shape=[
            jax.ShapeDtypeStruct((N_NODES, D), jnp.float32),
            jax.ShapeDtypeStruct((N_NODES, DH), jnp.float32),
            jax.ShapeDtypeStruct((N_NODES, DH), jnp.float32),
        ],
    )(degT, x, W1)


def _stage2_body(acc_ref, h1_ref, degT_ref, b1_ref, w2_ref,
                 h2_ref, hs_lo_ref, hs_hi_ref):
    dv = lax.rsqrt(1.0 + jnp.sum(degT_ref[...], axis=1, keepdims=True))
    a = acc_ref[0] + acc_ref[1]
    out1 = dv * a + dv * dv * h1_ref[...] + b1_ref[...]
    r = jnp.maximum(out1, 0.0)
    h2 = jnp.dot(r, w2_ref[...], preferred_element_type=jnp.float32)
    h2_ref[...] = h2
    hs = h2 * dv
    hs_lo_ref[...] = hs[:, :DH]
    hs_hi_ref[...] = hs[:, DH:]


def _stage2(acc1, h1, degT, b1, W2):
    return pl.pallas_call(
        _stage2_body,
        grid=(_GRID,),
        in_specs=[
            pl.BlockSpec((NC, _RB, D), lambda i: (0, i, 0)),
            pl.BlockSpec((_RB, D), lambda i: (i, 0)),
            pl.BlockSpec((_RB, NW), lambda i: (i, 0)),
            pl.BlockSpec((1, D), lambda i: (0, 0)),
            pl.BlockSpec((D, D), lambda i: (0, 0)),
        ],
        out_specs=[
            pl.BlockSpec((_RB, D), lambda i: (i, 0)),
            pl.BlockSpec((_RB, DH), lambda i: (i, 0)),
            pl.BlockSpec((_RB, DH), lambda i: (i, 0)),
        ],
        out_shape=[
            jax.ShapeDtypeStruct((N_NODES, D), jnp.float32),
            jax.ShapeDtypeStruct((N_NODES, DH), jnp.float32),
            jax.ShapeDtypeStruct((N_NODES, DH), jnp.float32),
        ],
    )(acc1, h1, degT, b1, W2)


def _stage3_body(acc_ref, h2_ref, degT_ref, b2_ref, out_ref):
    dv = lax.rsqrt(1.0 + jnp.sum(degT_ref[...], axis=1, keepdims=True))
    a = acc_ref[0] + acc_ref[1]
    out_ref[...] = dv * a + dv * dv * h2_ref[...] + b2_ref[...]


def _stage3(acc2, h2, degT, b2):
    return pl.pallas_call(
        _stage3_body,
        grid=(_GRID,),
        in_specs=[
            pl.BlockSpec((NC, _RB, D), lambda i: (0, i, 0)),
            pl.BlockSpec((_RB, D), lambda i: (i, 0)),
            pl.BlockSpec((_RB, NW), lambda i: (i, 0)),
            pl.BlockSpec((1, D), lambda i: (0, 0)),
        ],
        out_specs=pl.BlockSpec((_RB, D), lambda i: (i, 0)),
        out_shape=jax.ShapeDtypeStruct((N_NODES, D), jnp.float32),
    )(acc2, h2, degT, b2)


@jax.jit
def kernel(x, edge_index, edge_weight, W1, b1, W2, b2):
    src = edge_index[0].astype(jnp.int32)
    dst = edge_index[1].astype(jnp.int32)
    ew = edge_weight.astype(jnp.float32)

    src3 = src.reshape(NW, NCHUNK, CHUNK)
    dst3 = dst.reshape(NW, NCHUNK, CHUNK)
    ew2 = ew.reshape(NW, E_PER_W)

    deg_p = _deg_call(dst, ew)
    degT = deg_p.T
    h1, hs1_lo, hs1_hi = _stage1(degT, x, W1)
    acc1 = _agg_call(hs1_lo, hs1_hi, src3, dst3, ew2)
    h2, hs2_lo, hs2_hi = _stage2(acc1, h1, degT, b1.reshape(1, D), W2)
    acc2 = _agg_call(hs2_lo, hs2_hi, src3, dst3, ew2)
    return _stage3(acc2, h2, degT, b2.reshape(1, D))
